# baseline (speedup 1.0000x reference)
import functools

import jax
import jax.numpy as jnp
from jax import lax
from jax.experimental import pallas as pl
from jax.experimental.pallas import tpu as pltpu
from jax.experimental.pallas import tpu_sc as plsc


_B, _T, _M = 32, 576, 1024
_N = _B * _T
_N_SC = 6144
_N_TC = _N - _N_SC
_TC_ROWS = 3072


def _first_argmax(x):
    m = jnp.max(x, axis=-1, keepdims=True)
    iota = jax.lax.broadcasted_iota(jnp.int32, x.shape, 1)
    cand = jnp.where(x == m, iota, _M)
    return jnp.min(cand, axis=-1).astype(jnp.int32)


def _argmax_block(x_ref, idx_ref):
    idx_ref[...] = _first_argmax(x_ref[...])


def _tc_argmax_tail(x2):
    base = _N_TC // _TC_ROWS
    return pl.pallas_call(
        _argmax_block,
        grid=(_N_SC // _TC_ROWS,),
        in_specs=[pl.BlockSpec((_TC_ROWS, _M), lambda i: (i + base, 0))],
        out_specs=pl.BlockSpec((_TC_ROWS,), lambda i: (i,)),
        out_shape=jax.ShapeDtypeStruct((_N_SC,), jnp.int32),
        compiler_params=pltpu.CompilerParams(
            dimension_semantics=("arbitrary",),
        ),
    )(x2)


def _onehot_block(x_ref, o_ref):
    x = x_ref[...]
    idx = _first_argmax(x)
    iota = jax.lax.broadcasted_iota(jnp.int32, x.shape, 1)
    o_ref[...] = (iota == idx[:, None]).astype(x.dtype)


def _tc_fused_head(x2):
    return pl.pallas_call(
        _onehot_block,
        grid=(_N_TC // _TC_ROWS,),
        in_specs=[pl.BlockSpec((_TC_ROWS, _M), lambda i: (i, 0))],
        out_specs=pl.BlockSpec((_TC_ROWS, _M), lambda i: (i, 0)),
        out_shape=jax.ShapeDtypeStruct((_N_TC, _M), jnp.float32),
        compiler_params=pltpu.CompilerParams(
            dimension_semantics=("arbitrary",),
        ),
    )(x2)


_NC, _NS = 2, 16
_NW = _NC * _NS
_ROWS_PER_W = _N_SC // _NW
_RB = 32
_NBUF = 3
_NB = _ROWS_PER_W // _RB


def _sc_onehot_body(idx_hbm, out_hbm, idx_v, buf0, buf1, buf2, sem0, sem1, sem2):
    wid = lax.axis_index("s") * _NC + lax.axis_index("c")
    base = wid * _ROWS_PER_W
    pltpu.sync_copy(idx_hbm.at[pl.ds(base, _ROWS_PER_W)], idx_v)

    zero16 = jnp.zeros((16,), jnp.float32)
    one16 = jnp.ones((16,), jnp.float32)
    iota16 = lax.iota(jnp.int32, 16)
    bufs = (buf0, buf1, buf2)
    sems = (sem0, sem1, sem2)

    def zbody(i, _):
        r = i >> 6
        c = (i & 63) * 16
        buf0[r, pl.ds(c, 16)] = zero16
        buf1[r, pl.ds(c, 16)] = zero16
        buf2[r, pl.ds(c, 16)] = zero16
        return 0

    lax.fori_loop(0, _RB * (_M // 16), zbody, 0)

    def scatter(buf, b, val16):
        for g in range(_RB // 16):
            col = idx_v[pl.ds(b * _RB + g * 16, 16)]
            row = iota16 + (g * 16)
            plsc.store_scatter(buf, [row, col], val16)

    pending = [None] * _NBUF
    for b in range(_NB):
        k = b % _NBUF
        buf, sem = bufs[k], sems[k]
        if pending[k] is not None:
            pending[k].wait()
            scatter(buf, b - _NBUF, zero16)
        scatter(buf, b, one16)
        dst = out_hbm.at[pl.ds(base + b * _RB, _RB)]
        pending[k] = pltpu.async_copy(buf, dst, sem)
    for b in range(max(0, _NB - _NBUF), _NB):
        if pending[b % _NBUF] is not None:
            pending[b % _NBUF].wait()
            pending[b % _NBUF] = None


_sc_onehot = functools.partial(
    pl.kernel,
    mesh=plsc.VectorSubcoreMesh(core_axis_name="c", subcore_axis_name="s"),
    out_type=jax.ShapeDtypeStruct((_N_SC, _M), jnp.float32),
    scratch_types=[
        pltpu.VMEM((_ROWS_PER_W,), jnp.int32),
        pltpu.VMEM((_RB, _M), jnp.float32),
        pltpu.VMEM((_RB, _M), jnp.float32),
        pltpu.VMEM((_RB, _M), jnp.float32),
        pltpu.SemaphoreType.DMA,
        pltpu.SemaphoreType.DMA,
        pltpu.SemaphoreType.DMA,
    ],
    compiler_params=pltpu.CompilerParams(needs_layout_passes=False),
)(_sc_onehot_body)


def kernel(inputs):
    x2 = inputs.reshape(_N, _M)
    idx_sc = _tc_argmax_tail(x2)
    sc_out = _sc_onehot(idx_sc)
    tc_out = _tc_fused_head(x2)
    out = jnp.concatenate([tc_out, sc_out], axis=0)
    return out.reshape(_B, _T, _M)


# speedup vs baseline: 2.3588x; 2.3588x over previous
import jax
import jax.numpy as jnp
from jax.experimental import pallas as pl
from jax.experimental.pallas import tpu as pltpu


_ROWS_PER_BLOCK = 3072


def _onehot_argmax_block(x_ref, o_ref):
    x = x_ref[...]
    m = jnp.max(x, axis=-1, keepdims=True)
    iota = jax.lax.broadcasted_iota(jnp.int32, x.shape, 1)
    idx = jnp.min(jnp.where(x == m, iota, x.shape[-1]), axis=-1)
    o_ref[...] = (iota == idx[:, None]).astype(x.dtype)


def kernel(inputs):
    b, t, m = inputs.shape
    x2 = inputs.reshape(b * t, m)
    n = b * t
    out = pl.pallas_call(
        _onehot_argmax_block,
        grid=(n // _ROWS_PER_BLOCK,),
        in_specs=[pl.BlockSpec((_ROWS_PER_BLOCK, m), lambda i: (i, 0))],
        out_specs=pl.BlockSpec((_ROWS_PER_BLOCK, m), lambda i: (i, 0)),
        out_shape=jax.ShapeDtypeStruct((n, m), inputs.dtype),
        compiler_params=pltpu.CompilerParams(
            dimension_semantics=("parallel",),
        ),
    )(x2)
    return out.reshape(b, t, m)
